# baseline (speedup 1.0000x reference)
import functools

import jax
import jax.numpy as jnp
from jax import lax
from jax.experimental import pallas as pl
from jax.experimental.pallas import tpu as pltpu
from jax.experimental.pallas import tpu_sc as plsc

B = 4096
F = 13
NF = 26
V = 100000
D = 32
H1 = 512
H2 = 256
MLP_IN = D + NF * D

_NC = 2
_NS = 16
_NW = _NC * _NS

_COLS_PER_W = NF * D // _NW


def _sc_gather(tables_c, idx_fm):
    mesh = plsc.VectorSubcoreMesh(core_axis_name="c", subcore_axis_name="s")

    @functools.partial(
        pl.kernel,
        mesh=mesh,
        compiler_params=pltpu.CompilerParams(needs_layout_passes=False),
        out_type=jax.ShapeDtypeStruct((NF * D, B), jnp.float32),
        scratch_types=[
            pltpu.VMEM((V,), jnp.float32),
            pltpu.VMEM((B,), jnp.int32),
            pltpu.VMEM((2, B), jnp.float32),
            pltpu.SemaphoreType.DMA,
        ],
    )
    def k(table_hbm, idx_hbm, out_hbm, col_v, idx_v, res_v, sem):
        wid = lax.axis_index("s") * _NC + lax.axis_index("c")
        c0 = wid * _COLS_PER_W

        @pl.loop(0, _COLS_PER_W)
        def _(j):
            c = c0 + j
            f = c // D
            d = c - f * D
            pb = j % 2

            @pl.when(jnp.logical_or(j == 0, d == 0))
            def _():
                pltpu.sync_copy(idx_hbm.at[f], idx_v)

            pltpu.sync_copy(table_hbm.at[f, d], col_v)

            @pl.when(j >= 2)
            def _():
                pltpu.make_async_copy(
                    res_v.at[pb], out_hbm.at[c0], sem).wait()

            @pl.loop(0, B // 128)
            def _(b):
                for u in range(8):
                    o = b * 128 + u * 16
                    idx16 = idx_v[pl.ds(o, 16)]
                    res_v[pb, pl.ds(o, 16)] = plsc.load_gather(col_v, [idx16])

            pltpu.make_async_copy(res_v.at[pb], out_hbm.at[c], sem).start()

        pltpu.make_async_copy(res_v.at[0], out_hbm.at[c0], sem).wait()
        pltpu.make_async_copy(res_v.at[1], out_hbm.at[c0], sem).wait()

    return k(tables_c, idx_fm)


def _mlp_body(cont_ref, embT_ref, Wc_ref, bc_ref, W1c_ref, W1e_ref, b1_ref,
              W2_ref, b2_ref, Wo_ref, bo_ref, out_ref):
    xc = jnp.dot(cont_ref[...], Wc_ref[...],
                 preferred_element_type=jnp.float32) + bc_ref[...]
    x1 = jnp.dot(xc, W1c_ref[...], preferred_element_type=jnp.float32)
    xe = lax.dot_general(embT_ref[...], W1e_ref[...],
                         (((0,), (0,)), ((), ())),
                         preferred_element_type=jnp.float32)
    h1 = jnp.maximum(x1 + xe + b1_ref[...], 0.0)
    h2 = jnp.maximum(
        jnp.dot(h1, W2_ref[...], preferred_element_type=jnp.float32)
        + b2_ref[...], 0.0)
    o = jnp.dot(h2, Wo_ref[...], preferred_element_type=jnp.float32) + bo_ref[...]
    out_ref[...] = jax.nn.sigmoid(o)


def _tc_mlp(cont, embT, Wc, bc, W1c, W1e, b1, W2, b2, Wo, bo):
    blk = 512
    grid = (B // blk,)
    return pl.pallas_call(
        _mlp_body,
        grid=grid,
        in_specs=[
            pl.BlockSpec((blk, F), lambda i: (i, 0)),
            pl.BlockSpec((NF * D, blk), lambda i: (0, i)),
            pl.BlockSpec((F, D), lambda i: (0, 0)),
            pl.BlockSpec((1, D), lambda i: (0, 0)),
            pl.BlockSpec((D, H1), lambda i: (0, 0)),
            pl.BlockSpec((NF * D, H1), lambda i: (0, 0)),
            pl.BlockSpec((1, H1), lambda i: (0, 0)),
            pl.BlockSpec((H1, H2), lambda i: (0, 0)),
            pl.BlockSpec((1, H2), lambda i: (0, 0)),
            pl.BlockSpec((H2, 1), lambda i: (0, 0)),
            pl.BlockSpec((1, 1), lambda i: (0, 0)),
        ],
        out_specs=pl.BlockSpec((blk, 1), lambda i: (i, 0)),
        out_shape=jax.ShapeDtypeStruct((B, 1), jnp.float32),
    )(cont, embT, Wc, bc, W1c, W1e, b1, W2, b2, Wo, bo)


def kernel(continuous_features, categorical_features, tables, Wc, bc, W1, b1,
           W2, b2, Wo, bo):
    idx_fm = categorical_features.astype(jnp.int32).T
    tables_c = tables.transpose(0, 2, 1)
    embT = _sc_gather(tables_c, idx_fm)
    return _tc_mlp(continuous_features, embT,
                   Wc, bc.reshape(1, D),
                   W1[:D], W1[D:], b1.reshape(1, H1),
                   W2, b2.reshape(1, H2),
                   Wo, bo.reshape(1, 1))


# speedup vs baseline: 1.1452x; 1.1452x over previous
import functools

import jax
import jax.numpy as jnp
from jax import lax
from jax.experimental import pallas as pl
from jax.experimental.pallas import tpu as pltpu
from jax.experimental.pallas import tpu_sc as plsc

B = 4096
F = 13
NF = 26
V = 100000
D = 32
H1 = 512
H2 = 256
MLP_IN = D + NF * D

_NC = 2
_NS = 16
_NW = _NC * _NS

_COLS_PER_W = NF * D // _NW


def _sc_gather(tables_c, idx_fm):
    mesh = plsc.VectorSubcoreMesh(core_axis_name="c", subcore_axis_name="s")

    @functools.partial(
        pl.kernel,
        mesh=mesh,
        compiler_params=pltpu.CompilerParams(needs_layout_passes=False),
        out_type=jax.ShapeDtypeStruct((NF * D, B), jnp.float32),
        scratch_types=[
            pltpu.VMEM((V,), jnp.float32),
            pltpu.VMEM((B,), jnp.int32),
            pltpu.VMEM((B,), jnp.float32),
        ],
    )
    def k(table_hbm, idx_hbm, out_hbm, col_v, idx_v, res_v):
        wid = lax.axis_index("s") * _NC + lax.axis_index("c")
        c0 = wid * _COLS_PER_W

        @pl.loop(0, _COLS_PER_W)
        def _(j):
            c = c0 + j
            f = c // D
            d = c - f * D

            @pl.when(jnp.logical_or(j == 0, d == 0))
            def _():
                pltpu.sync_copy(idx_hbm.at[f], idx_v)

            pltpu.sync_copy(table_hbm.at[f, d], col_v)

            @pl.loop(0, B // 64)
            def _(b):
                for u in range(4):
                    o = b * 64 + u * 16
                    idx16 = idx_v[pl.ds(o, 16)]
                    res_v[pl.ds(o, 16)] = plsc.load_gather(col_v, [idx16])

            pltpu.sync_copy(res_v, out_hbm.at[c])

    return k(tables_c, idx_fm)


def _mlp_body(cont_ref, embT_ref, Wc_ref, bc_ref, W1c_ref, W1e_ref, b1_ref,
              W2_ref, b2_ref, Wo_ref, bo_ref, out_ref):
    xc = jnp.dot(cont_ref[...], Wc_ref[...],
                 preferred_element_type=jnp.float32) + bc_ref[...]
    x1 = jnp.dot(xc, W1c_ref[...], preferred_element_type=jnp.float32)
    xe = lax.dot_general(embT_ref[...], W1e_ref[...],
                         (((0,), (0,)), ((), ())),
                         preferred_element_type=jnp.float32)
    h1 = jnp.maximum(x1 + xe + b1_ref[...], 0.0)
    h2 = jnp.maximum(
        jnp.dot(h1, W2_ref[...], preferred_element_type=jnp.float32)
        + b2_ref[...], 0.0)
    o = jnp.dot(h2, Wo_ref[...], preferred_element_type=jnp.float32) + bo_ref[...]
    out_ref[...] = jax.nn.sigmoid(o)


def _tc_mlp(cont, embT, Wc, bc, W1c, W1e, b1, W2, b2, Wo, bo):
    blk = 512
    grid = (B // blk,)
    return pl.pallas_call(
        _mlp_body,
        grid=grid,
        in_specs=[
            pl.BlockSpec((blk, F), lambda i: (i, 0)),
            pl.BlockSpec((NF * D, blk), lambda i: (0, i)),
            pl.BlockSpec((F, D), lambda i: (0, 0)),
            pl.BlockSpec((1, D), lambda i: (0, 0)),
            pl.BlockSpec((D, H1), lambda i: (0, 0)),
            pl.BlockSpec((NF * D, H1), lambda i: (0, 0)),
            pl.BlockSpec((1, H1), lambda i: (0, 0)),
            pl.BlockSpec((H1, H2), lambda i: (0, 0)),
            pl.BlockSpec((1, H2), lambda i: (0, 0)),
            pl.BlockSpec((H2, 1), lambda i: (0, 0)),
            pl.BlockSpec((1, 1), lambda i: (0, 0)),
        ],
        out_specs=pl.BlockSpec((blk, 1), lambda i: (i, 0)),
        out_shape=jax.ShapeDtypeStruct((B, 1), jnp.float32),
    )(cont, embT, Wc, bc, W1c, W1e, b1, W2, b2, Wo, bo)


def kernel(continuous_features, categorical_features, tables, Wc, bc, W1, b1,
           W2, b2, Wo, bo):
    idx_fm = categorical_features.astype(jnp.int32).T
    tables_c = tables.transpose(0, 2, 1)
    embT = _sc_gather(tables_c, idx_fm)
    return _tc_mlp(continuous_features, embT,
                   Wc, bc.reshape(1, D),
                   W1[:D], W1[D:], b1.reshape(1, H1),
                   W2, b2.reshape(1, H2),
                   Wo, bo.reshape(1, 1))
